# baseline (speedup 1.0000x reference)
import functools

import jax
import jax.numpy as jnp
from jax import lax
from jax.experimental import pallas as pl
from jax.experimental.pallas import tpu as pltpu
from jax.experimental.pallas import tpu_sc as plsc

N = 10000
NPAD = 10240
E = 640000
D = 128
P = 4096

NC, NS = 2, 16
NW = NC * NS
CH = 128
NROW = E // CH
GRP = 8
NGROUP = NROW // GRP
GREM = NGROUP % NW
MAXG = NGROUP // NW + 1
NBUF = 4
RPT = NPAD // NS

_MESH = plsc.VectorSubcoreMesh(
    core_axis_name="c", subcore_axis_name="s", num_cores=NC, num_subcores=NS)


def _worker_groups(wid):
    return jnp.where(wid < GREM, NGROUP // NW + 1, NGROUP // NW)


def _agg_body(hd, src2, dst2, zeros128, out,
              idx_s, idx_d, r0, r1, sem_i, g0, g1, acc):
    cid = lax.axis_index("c")
    sid = lax.axis_index("s")
    wid = sid * NC + cid
    sl = pl.ds(sid * RPT, RPT)
    ng = _worker_groups(wid)

    pltpu.sync_copy(zeros128.at[sl], acc.at[sl])
    gr0 = wid * GRP
    pltpu.sync_copy(src2.at[pl.ds(gr0, GRP)], idx_s.at[pl.ds(0, GRP)])
    pltpu.sync_copy(dst2.at[pl.ds(gr0, GRP)], idx_d.at[pl.ds(0, GRP)])
    plsc.subcore_barrier()
    pltpu.async_copy(hd.at[idx_s.at[0]], r0, g0)

    def group_body(g, c):
        half = (g % 2) * GRP
        nxt_half = ((g + 1) % 2) * GRP
        have_next = g + 1 < ng

        @pl.when(have_next)
        def _prefetch_idx():
            gr = (wid + (g + 1) * NW) * GRP
            pltpu.async_copy(src2.at[pl.ds(gr, GRP)],
                             idx_s.at[pl.ds(nxt_half, GRP)], sem_i)
            pltpu.async_copy(dst2.at[pl.ds(gr, GRP)],
                             idx_d.at[pl.ds(nxt_half, GRP)], sem_i)

        for j in range(GRP):
            rr, gg = (r0, g0) if j % 2 == 0 else (r1, g1)
            orr, ogg = (r1, g1) if j % 2 == 0 else (r0, g0)
            row = half + j
            pltpu.make_async_copy(zeros128.at[pl.ds(0, CH)], rr, gg).wait()
            if j < GRP - 1:
                pltpu.async_copy(hd.at[idx_s.at[row + 1]], orr, ogg)
            else:
                @pl.when(have_next)
                def _next_group_gather():
                    pltpu.make_async_copy(
                        src2.at[pl.ds(0, GRP)], idx_s.at[pl.ds(0, GRP)], sem_i).wait()
                    pltpu.make_async_copy(
                        dst2.at[pl.ds(0, GRP)], idx_d.at[pl.ds(0, GRP)], sem_i).wait()
                    pltpu.async_copy(hd.at[idx_s.at[nxt_half]], orr, ogg)
            pltpu.sync_copy(rr, acc.at[idx_d.at[row]], add=True)
        return c

    lax.fori_loop(0, ng, group_body, 0)
    plsc.subcore_barrier()
    pltpu.sync_copy(acc.at[sl], out.at[cid, sl])


_agg_call = pl.kernel(
    _agg_body,
    out_type=jax.ShapeDtypeStruct((NC, NPAD, D), jnp.float32),
    mesh=_MESH,
    scratch_types=[
        pltpu.VMEM((2 * GRP, CH), jnp.int32),
        pltpu.VMEM((2 * GRP, CH), jnp.int32),
        pltpu.VMEM((CH, D), jnp.float32),
        pltpu.VMEM((CH, D), jnp.float32),
        pltpu.SemaphoreType.DMA,
        pltpu.SemaphoreType.DMA,
        pltpu.SemaphoreType.DMA,
        pltpu.VMEM_SHARED((NPAD, D), jnp.float32),
    ],
)

GCHUNK = 128
GPW = 2 * P // NW


def _pair_gather_body(h2, idxflat, out, gidx, grows, sem):
    cid = lax.axis_index("c")
    sid = lax.axis_index("s")
    wid = sid * NC + cid
    base0 = wid * GPW

    def body(i, carry):
        base = base0 + i * GCHUNK
        pltpu.sync_copy(idxflat.at[pl.ds(base, GCHUNK)], gidx)
        pltpu.async_copy(h2.at[gidx], grows, sem).wait()
        pltpu.sync_copy(grows, out.at[pl.ds(base, GCHUNK)])
        return carry

    lax.fori_loop(0, GPW // GCHUNK, body, 0)


_pair_gather_call = pl.kernel(
    _pair_gather_body,
    out_type=jax.ShapeDtypeStruct((2 * P, D), jnp.float32),
    mesh=_MESH,
    scratch_types=[
        pltpu.VMEM((GCHUNK,), jnp.int32),
        pltpu.VMEM((GCHUNK, D), jnp.float32),
        pltpu.SemaphoreType.DMA,
    ],
)


def _dinv(dpa, dpb):
    deg = dpa + dpb + 1.0
    return lax.rsqrt(deg)


def _mm_raw_body(x_ref, w_ref, o_ref):
    o_ref[...] = jnp.dot(x_ref[...], w_ref[...], preferred_element_type=jnp.float32)


def _scale_body(dpa_ref, dpb_ref, h_ref, o_ref):
    o_ref[...] = h_ref[...] * _dinv(dpa_ref[...], dpb_ref[...])


def _mid_body(dpa_ref, dpb_ref, pa_ref, pb_ref, hd_ref, b_ref, w_ref, o_ref):
    dinv = _dinv(dpa_ref[...], dpb_ref[...])
    agg = pa_ref[...] + pb_ref[...] + hd_ref[...]
    h1 = jnp.maximum(agg * dinv + b_ref[...], 0.0)
    o_ref[...] = jnp.dot(h1, w_ref[...], preferred_element_type=jnp.float32) * dinv


def _final_body(dpa_ref, dpb_ref, qa_ref, qb_ref, hd_ref, b_ref, o_ref):
    dinv = _dinv(dpa_ref[...], dpb_ref[...])
    agg = qa_ref[...] + qb_ref[...] + hd_ref[...]
    o_ref[...] = agg * dinv + b_ref[...]


BR = 1024


def _full(shape):
    return pl.BlockSpec(shape, lambda i: (0,) * len(shape))


def _row_call(body, n_rows_in, w_shapes, nrows=NPAD):
    in_specs = (
        [pl.BlockSpec((BR, D), lambda i: (i, 0))] * n_rows_in
        + [_full(sh) for sh in w_shapes]
    )
    return pl.pallas_call(
        body,
        grid=(nrows // BR,),
        in_specs=in_specs,
        out_specs=pl.BlockSpec((BR, D), lambda i: (i, 0)),
        out_shape=jax.ShapeDtypeStruct((nrows, D), jnp.float32),
    )


def _rest_body(scg_ref, esm_ref, gpt_ref, wscg_ref, wesm_ref, wgpt_ref, cb1_ref, o_ref):
    z = jnp.dot(scg_ref[...], wscg_ref[...], preferred_element_type=jnp.float32)
    z += jnp.dot(esm_ref[...], wesm_ref[...], preferred_element_type=jnp.float32)
    z += jnp.dot(gpt_ref[...], wgpt_ref[...], preferred_element_type=jnp.float32)
    o_ref[...] = z + cb1_ref[...]


def _mlp_body(ne1_ref, ne2_ref, rest_ref, wg1_ref, wg2_ref,
              w2_ref, cb2_ref, w3_ref, cb3_ref, o_ref):
    z1 = jnp.dot(ne1_ref[...], wg1_ref[...], preferred_element_type=jnp.float32)
    z1 += jnp.dot(ne2_ref[...], wg2_ref[...], preferred_element_type=jnp.float32)
    z1 = jnp.maximum(z1 + rest_ref[...], 0.0)
    z2 = jnp.maximum(
        jnp.dot(z1, w2_ref[...], preferred_element_type=jnp.float32) + cb2_ref[...], 0.0)
    o_ref[...] = jnp.dot(z2, w3_ref[...], preferred_element_type=jnp.float32) + cb3_ref[...]


MBR = 512


def _rest_call(scgw, esmw, gptw):
    in_specs = (
        [pl.BlockSpec((MBR, scgw), lambda i: (i, 0)),
         pl.BlockSpec((MBR, esmw), lambda i: (i, 0)),
         pl.BlockSpec((MBR, gptw), lambda i: (i, 0))]
        + [_full(sh) for sh in [(scgw, D), (esmw, D), (gptw, D), (1, D)]]
    )
    return pl.pallas_call(
        _rest_body,
        grid=(P // MBR,),
        in_specs=in_specs,
        out_specs=pl.BlockSpec((MBR, D), lambda i: (i, 0)),
        out_shape=jax.ShapeDtypeStruct((P, D), jnp.float32),
    )


def _mlp_call():
    in_specs = (
        [pl.BlockSpec((MBR, D), lambda i: (i, 0))] * 3
        + [_full(sh) for sh in [(D, D), (D, D), (D, D), (1, D), (D, D), (1, D)]]
    )
    return pl.pallas_call(
        _mlp_body,
        grid=(P // MBR,),
        in_specs=in_specs,
        out_specs=pl.BlockSpec((MBR, D), lambda i: (i, 0)),
        out_shape=jax.ShapeDtypeStruct((P, D), jnp.float32),
    )


def kernel(x, edge_index, scg_pair, gpt_pair, esm_pair, pair_idx,
           W1, b1, W2, b2, cW1, cb1, cW2, cb2, cW3, cb3):
    f32 = jnp.float32
    x_pad = jnp.pad(x, ((0, NPAD - N), (0, 0)))
    z128 = jnp.zeros((NPAD, D), f32)
    ones128 = jnp.ones((NPAD, D), f32)
    src2 = edge_index[0].reshape(NROW, CH)
    dst2 = edge_index[1].reshape(NROW, CH)

    scgw = scg_pair.shape[1]
    esmw = esm_pair.shape[1]
    gptw = gpt_pair.shape[1]
    wg1 = cW1[:D]
    wg2 = cW1[D:2 * D]
    wscg = cW1[2 * D:2 * D + scgw]
    wesm = cW1[2 * D + scgw:2 * D + scgw + esmw]
    wgpt = cW1[2 * D + scgw + esmw:]
    cW2p = jnp.pad(cW2, ((0, 0), (0, D - cW2.shape[1])))
    cb2p = jnp.pad(cb2, (0, D - cb2.shape[0])).reshape(1, D)
    cW3p = jnp.pad(cW3, ((0, D - cW3.shape[0]), (0, D - cW3.shape[1])))
    cb3p = jnp.pad(cb3, (0, D - cb3.shape[0])).reshape(1, D)

    deg_parts = _agg_call(ones128, dst2, dst2, z128)
    h1raw = _row_call(_mm_raw_body, 1, [(D, D)])(x_pad, W1)
    rest = _rest_call(scgw, esmw, gptw)(
        scg_pair, esm_pair, gpt_pair, wscg, wesm, wgpt, cb1.reshape(1, D))

    dpa, dpb = deg_parts[0], deg_parts[1]
    hd1 = _row_call(_scale_body, 3, [])(dpa, dpb, h1raw)
    p = _agg_call(hd1, src2, dst2, z128)
    hd2 = _row_call(_mid_body, 5, [(1, D), (D, D)])(
        dpa, dpb, p[0], p[1], hd1, b1.reshape(1, D), W2)
    q = _agg_call(hd2, src2, dst2, z128)
    h2 = _row_call(_final_body, 5, [(1, D)])(
        dpa, dpb, q[0], q[1], hd2, b2.reshape(1, D))

    idx_flat = jnp.transpose(pair_idx).reshape(2 * P)
    pg = _pair_gather_call(h2, idx_flat)
    ne1, ne2 = pg[:P], pg[P:]

    out = _mlp_call()(ne1, ne2, rest, wg1, wg2, cW2p, cb2p, cW3p, cb3p)
    return out[:, :cW3.shape[1]]


# speedup vs baseline: 1.0035x; 1.0035x over previous
import functools

import jax
import jax.numpy as jnp
from jax import lax
from jax.experimental import pallas as pl
from jax.experimental.pallas import tpu as pltpu
from jax.experimental.pallas import tpu_sc as plsc

N = 10000
NPAD = 10240
E = 640000
D = 128
P = 4096

NC, NS = 2, 16
NW = NC * NS
CH = 128
NROW = E // CH
GRP = 8
NGROUP = NROW // GRP
GREM = NGROUP % NW
MAXG = NGROUP // NW + 1
NBUF = 4
RPT = NPAD // NS

_MESH = plsc.VectorSubcoreMesh(
    core_axis_name="c", subcore_axis_name="s", num_cores=NC, num_subcores=NS)


def _worker_groups(wid):
    return jnp.where(wid < GREM, NGROUP // NW + 1, NGROUP // NW)


def _agg_body(hd, src2, dst2, zeros128, out,
              idx_s, idx_d, r0, r1, sem_i, g0, g1, s0, s1, acc):
    cid = lax.axis_index("c")
    sid = lax.axis_index("s")
    wid = sid * NC + cid
    sl = pl.ds(sid * RPT, RPT)
    ng = _worker_groups(wid)

    pltpu.sync_copy(zeros128.at[sl], acc.at[sl])
    gr0 = wid * GRP
    pltpu.sync_copy(src2.at[pl.ds(gr0, GRP)], idx_s.at[pl.ds(0, GRP)])
    pltpu.sync_copy(dst2.at[pl.ds(gr0, GRP)], idx_d.at[pl.ds(0, GRP)])
    plsc.subcore_barrier()
    pltpu.async_copy(hd.at[idx_s.at[0]], r0, g0)

    def group_body(g, c):
        half = (g % 2) * GRP
        nxt_half = ((g + 1) % 2) * GRP
        have_next = g + 1 < ng

        @pl.when(have_next)
        def _prefetch_idx():
            gr = (wid + (g + 1) * NW) * GRP
            pltpu.async_copy(src2.at[pl.ds(gr, GRP)],
                             idx_s.at[pl.ds(nxt_half, GRP)], sem_i)
            pltpu.async_copy(dst2.at[pl.ds(gr, GRP)],
                             idx_d.at[pl.ds(nxt_half, GRP)], sem_i)

        prev_scatter = None
        for j in range(GRP):
            rr, gg = (r0, g0) if j % 2 == 0 else (r1, g1)
            orr, ogg = (r1, g1) if j % 2 == 0 else (r0, g0)
            sc = s0 if j % 2 == 0 else s1
            row = half + j
            pltpu.make_async_copy(zeros128.at[pl.ds(0, CH)], rr, gg).wait()
            if j < GRP - 1:
                pltpu.async_copy(hd.at[idx_s.at[row + 1]], orr, ogg)
            else:
                @pl.when(have_next)
                def _next_group_gather():
                    pltpu.make_async_copy(
                        src2.at[pl.ds(0, GRP)], idx_s.at[pl.ds(0, GRP)], sem_i).wait()
                    pltpu.make_async_copy(
                        dst2.at[pl.ds(0, GRP)], idx_d.at[pl.ds(0, GRP)], sem_i).wait()
                    pltpu.async_copy(hd.at[idx_s.at[nxt_half]], orr, ogg)
            if prev_scatter is not None:
                prev_scatter.wait()
            if j < GRP - 1:
                prev_scatter = pltpu.async_copy(
                    rr, acc.at[idx_d.at[row]], sc, add=True)
            else:
                pltpu.sync_copy(rr, acc.at[idx_d.at[row]], add=True)
        return c

    lax.fori_loop(0, ng, group_body, 0)
    plsc.subcore_barrier()
    pltpu.sync_copy(acc.at[sl], out.at[cid, sl])


_agg_call = pl.kernel(
    _agg_body,
    out_type=jax.ShapeDtypeStruct((NC, NPAD, D), jnp.float32),
    mesh=_MESH,
    scratch_types=[
        pltpu.VMEM((2 * GRP, CH), jnp.int32),
        pltpu.VMEM((2 * GRP, CH), jnp.int32),
        pltpu.VMEM((CH, D), jnp.float32),
        pltpu.VMEM((CH, D), jnp.float32),
        pltpu.SemaphoreType.DMA,
        pltpu.SemaphoreType.DMA,
        pltpu.SemaphoreType.DMA,
        pltpu.SemaphoreType.DMA,
        pltpu.SemaphoreType.DMA,
        pltpu.VMEM_SHARED((NPAD, D), jnp.float32),
    ],
)

GCHUNK = 128
GPW = 2 * P // NW


def _pair_gather_body(h2, idxflat, out, gidx, grows, sem):
    cid = lax.axis_index("c")
    sid = lax.axis_index("s")
    wid = sid * NC + cid
    base0 = wid * GPW

    def body(i, carry):
        base = base0 + i * GCHUNK
        pltpu.sync_copy(idxflat.at[pl.ds(base, GCHUNK)], gidx)
        pltpu.async_copy(h2.at[gidx], grows, sem).wait()
        pltpu.sync_copy(grows, out.at[pl.ds(base, GCHUNK)])
        return carry

    lax.fori_loop(0, GPW // GCHUNK, body, 0)


_pair_gather_call = pl.kernel(
    _pair_gather_body,
    out_type=jax.ShapeDtypeStruct((2 * P, D), jnp.float32),
    mesh=_MESH,
    scratch_types=[
        pltpu.VMEM((GCHUNK,), jnp.int32),
        pltpu.VMEM((GCHUNK, D), jnp.float32),
        pltpu.SemaphoreType.DMA,
    ],
)


def _dinv(dpa, dpb):
    deg = dpa + dpb + 1.0
    return lax.rsqrt(deg)


def _mm_scale_body(dpa_ref, dpb_ref, x_ref, w_ref, o_ref):
    dinv = _dinv(dpa_ref[...], dpb_ref[...])
    h = jnp.dot(x_ref[...], w_ref[...], preferred_element_type=jnp.float32)
    o_ref[...] = h * dinv


def _mid_body(dpa_ref, dpb_ref, pa_ref, pb_ref, hd_ref, b_ref, w_ref, o_ref):
    dinv = _dinv(dpa_ref[...], dpb_ref[...])
    agg = pa_ref[...] + pb_ref[...] + hd_ref[...]
    h1 = jnp.maximum(agg * dinv + b_ref[...], 0.0)
    o_ref[...] = jnp.dot(h1, w_ref[...], preferred_element_type=jnp.float32) * dinv


def _final_body(dpa_ref, dpb_ref, qa_ref, qb_ref, hd_ref, b_ref, o_ref):
    dinv = _dinv(dpa_ref[...], dpb_ref[...])
    agg = qa_ref[...] + qb_ref[...] + hd_ref[...]
    o_ref[...] = agg * dinv + b_ref[...]


BR = 1024


def _full(shape):
    return pl.BlockSpec(shape, lambda i: (0,) * len(shape))


def _row_call(body, n_rows_in, w_shapes, nrows=NPAD):
    in_specs = (
        [pl.BlockSpec((BR, D), lambda i: (i, 0))] * n_rows_in
        + [_full(sh) for sh in w_shapes]
    )
    return pl.pallas_call(
        body,
        grid=(nrows // BR,),
        in_specs=in_specs,
        out_specs=pl.BlockSpec((BR, D), lambda i: (i, 0)),
        out_shape=jax.ShapeDtypeStruct((nrows, D), jnp.float32),
    )


def _rest_body(scg_ref, esm_ref, gpt_ref, wscg_ref, wesm_ref, wgpt_ref, cb1_ref, o_ref):
    z = jnp.dot(scg_ref[...], wscg_ref[...], preferred_element_type=jnp.float32)
    z += jnp.dot(esm_ref[...], wesm_ref[...], preferred_element_type=jnp.float32)
    z += jnp.dot(gpt_ref[...], wgpt_ref[...], preferred_element_type=jnp.float32)
    o_ref[...] = z + cb1_ref[...]


def _mlp_body(ne1_ref, ne2_ref, rest_ref, wg1_ref, wg2_ref,
              w2_ref, cb2_ref, w3_ref, cb3_ref, o_ref):
    z1 = jnp.dot(ne1_ref[...], wg1_ref[...], preferred_element_type=jnp.float32)
    z1 += jnp.dot(ne2_ref[...], wg2_ref[...], preferred_element_type=jnp.float32)
    z1 = jnp.maximum(z1 + rest_ref[...], 0.0)
    z2 = jnp.maximum(
        jnp.dot(z1, w2_ref[...], preferred_element_type=jnp.float32) + cb2_ref[...], 0.0)
    o_ref[...] = jnp.dot(z2, w3_ref[...], preferred_element_type=jnp.float32) + cb3_ref[...]


MBR = 512


def _rest_call(scgw, esmw, gptw):
    in_specs = (
        [pl.BlockSpec((MBR, scgw), lambda i: (i, 0)),
         pl.BlockSpec((MBR, esmw), lambda i: (i, 0)),
         pl.BlockSpec((MBR, gptw), lambda i: (i, 0))]
        + [_full(sh) for sh in [(scgw, D), (esmw, D), (gptw, D), (1, D)]]
    )
    return pl.pallas_call(
        _rest_body,
        grid=(P // MBR,),
        in_specs=in_specs,
        out_specs=pl.BlockSpec((MBR, D), lambda i: (i, 0)),
        out_shape=jax.ShapeDtypeStruct((P, D), jnp.float32),
    )


def _mlp_call():
    in_specs = (
        [pl.BlockSpec((MBR, D), lambda i: (i, 0))] * 3
        + [_full(sh) for sh in [(D, D), (D, D), (D, D), (1, D), (D, D), (1, D)]]
    )
    return pl.pallas_call(
        _mlp_body,
        grid=(P // MBR,),
        in_specs=in_specs,
        out_specs=pl.BlockSpec((MBR, D), lambda i: (i, 0)),
        out_shape=jax.ShapeDtypeStruct((P, D), jnp.float32),
    )


def kernel(x, edge_index, scg_pair, gpt_pair, esm_pair, pair_idx,
           W1, b1, W2, b2, cW1, cb1, cW2, cb2, cW3, cb3):
    f32 = jnp.float32
    x_pad = jnp.pad(x, ((0, NPAD - N), (0, 0)))
    z128 = jnp.zeros((NPAD, D), f32)
    ones128 = jnp.ones((NPAD, D), f32)
    src2 = edge_index[0].reshape(NROW, CH)
    dst2 = edge_index[1].reshape(NROW, CH)

    scgw = scg_pair.shape[1]
    esmw = esm_pair.shape[1]
    gptw = gpt_pair.shape[1]
    wg1 = cW1[:D]
    wg2 = cW1[D:2 * D]
    wscg = cW1[2 * D:2 * D + scgw]
    wesm = cW1[2 * D + scgw:2 * D + scgw + esmw]
    wgpt = cW1[2 * D + scgw + esmw:]
    cW2p = jnp.pad(cW2, ((0, 0), (0, D - cW2.shape[1])))
    cb2p = jnp.pad(cb2, (0, D - cb2.shape[0])).reshape(1, D)
    cW3p = jnp.pad(cW3, ((0, D - cW3.shape[0]), (0, D - cW3.shape[1])))
    cb3p = jnp.pad(cb3, (0, D - cb3.shape[0])).reshape(1, D)

    deg_parts = _agg_call(ones128, dst2, dst2, z128)
    rest = _rest_call(scgw, esmw, gptw)(
        scg_pair, esm_pair, gpt_pair, wscg, wesm, wgpt, cb1.reshape(1, D))

    dpa, dpb = deg_parts[0], deg_parts[1]
    hd1 = _row_call(_mm_scale_body, 3, [(D, D)])(dpa, dpb, x_pad, W1)
    p = _agg_call(hd1, src2, dst2, z128)
    hd2 = _row_call(_mid_body, 5, [(1, D), (D, D)])(
        dpa, dpb, p[0], p[1], hd1, b1.reshape(1, D), W2)
    q = _agg_call(hd2, src2, dst2, z128)
    h2 = _row_call(_final_body, 5, [(1, D)])(
        dpa, dpb, q[0], q[1], hd2, b2.reshape(1, D))

    idx_flat = jnp.transpose(pair_idx).reshape(2 * P)
    pg = _pair_gather_call(h2, idx_flat)
    ne1, ne2 = pg[:P], pg[P:]

    out = _mlp_call()(ne1, ne2, rest, wg1, wg2, cW2p, cb2p, cW3p, cb3p)
    return out[:, :cW3.shape[1]]


# speedup vs baseline: 1.1813x; 1.1772x over previous
import functools

import jax
import jax.numpy as jnp
from jax import lax
from jax.experimental import pallas as pl
from jax.experimental.pallas import tpu as pltpu
from jax.experimental.pallas import tpu_sc as plsc

N = 10000
NPAD = 10240
E = 640000
D = 128
P = 4096

NC, NS = 2, 16
NW = NC * NS
CH = 128
NROW = E // CH
GRP = 8
NGROUP = NROW // GRP
GREM = NGROUP % NW
MAXG = NGROUP // NW + 1
NBUF = 4
RPT = NPAD // NS

_MESH = plsc.VectorSubcoreMesh(
    core_axis_name="c", subcore_axis_name="s", num_cores=NC, num_subcores=NS)


def _worker_groups(wid):
    return jnp.where(wid < GREM, NGROUP // NW + 1, NGROUP // NW)


def _agg_body(hd, src2, dst2, zeros128, out,
              idx_s, idx_d, r0, r1, sem_i, g0, g1, s0, s1, acc):
    cid = lax.axis_index("c")
    sid = lax.axis_index("s")
    wid = sid * NC + cid
    sl = pl.ds(sid * RPT, RPT)
    ng = _worker_groups(wid)

    pltpu.sync_copy(zeros128.at[sl], acc.at[sl])
    gr0 = wid * GRP
    pltpu.sync_copy(src2.at[pl.ds(gr0, GRP)], idx_s.at[pl.ds(0, GRP)])
    pltpu.sync_copy(dst2.at[pl.ds(gr0, GRP)], idx_d.at[pl.ds(0, GRP)])
    plsc.subcore_barrier()
    pltpu.async_copy(hd.at[idx_s.at[0]], r0, g0)

    def group_body(g, c):
        half = (g % 2) * GRP
        nxt_half = ((g + 1) % 2) * GRP
        have_next = g + 1 < ng

        @pl.when(have_next)
        def _prefetch_idx():
            gr = (wid + (g + 1) * NW) * GRP
            pltpu.async_copy(src2.at[pl.ds(gr, GRP)],
                             idx_s.at[pl.ds(nxt_half, GRP)], sem_i)
            pltpu.async_copy(dst2.at[pl.ds(gr, GRP)],
                             idx_d.at[pl.ds(nxt_half, GRP)], sem_i)

        prev_scatter = None
        for j in range(GRP):
            rr, gg = (r0, g0) if j % 2 == 0 else (r1, g1)
            orr, ogg = (r1, g1) if j % 2 == 0 else (r0, g0)
            sc = s0 if j % 2 == 0 else s1
            row = half + j
            if prev_scatter is not None:
                prev_scatter.wait()
            if j < GRP - 1:
                pltpu.async_copy(hd.at[idx_s.at[row + 1]], orr, ogg)
            else:
                @pl.when(have_next)
                def _next_group_gather():
                    pltpu.make_async_copy(
                        src2.at[pl.ds(0, GRP)], idx_s.at[pl.ds(0, GRP)], sem_i).wait()
                    pltpu.make_async_copy(
                        dst2.at[pl.ds(0, GRP)], idx_d.at[pl.ds(0, GRP)], sem_i).wait()
                    pltpu.async_copy(hd.at[idx_s.at[nxt_half]], orr, ogg)
            pltpu.make_async_copy(zeros128.at[pl.ds(0, CH)], rr, gg).wait()
            if j < GRP - 1:
                prev_scatter = pltpu.async_copy(
                    rr, acc.at[idx_d.at[row]], sc, add=True)
            else:
                pltpu.sync_copy(rr, acc.at[idx_d.at[row]], add=True)
        return c

    lax.fori_loop(0, ng, group_body, 0)
    plsc.subcore_barrier()
    pltpu.sync_copy(acc.at[sl], out.at[cid, sl])


_agg_call = pl.kernel(
    _agg_body,
    out_type=jax.ShapeDtypeStruct((NC, NPAD, D), jnp.float32),
    mesh=_MESH,
    scratch_types=[
        pltpu.VMEM((2 * GRP, CH), jnp.int32),
        pltpu.VMEM((2 * GRP, CH), jnp.int32),
        pltpu.VMEM((CH, D), jnp.float32),
        pltpu.VMEM((CH, D), jnp.float32),
        pltpu.SemaphoreType.DMA,
        pltpu.SemaphoreType.DMA,
        pltpu.SemaphoreType.DMA,
        pltpu.SemaphoreType.DMA,
        pltpu.SemaphoreType.DMA,
        pltpu.VMEM_SHARED((NPAD, D), jnp.float32),
    ],
)

GCHUNK = 128
GPW = 2 * P // NW


def _pair_gather_body(h2, idxflat, out, gidx, grows, sem):
    cid = lax.axis_index("c")
    sid = lax.axis_index("s")
    wid = sid * NC + cid
    base0 = wid * GPW

    def body(i, carry):
        base = base0 + i * GCHUNK
        pltpu.sync_copy(idxflat.at[pl.ds(base, GCHUNK)], gidx)
        pltpu.async_copy(h2.at[gidx], grows, sem).wait()
        pltpu.sync_copy(grows, out.at[pl.ds(base, GCHUNK)])
        return carry

    lax.fori_loop(0, GPW // GCHUNK, body, 0)


_pair_gather_call = pl.kernel(
    _pair_gather_body,
    out_type=jax.ShapeDtypeStruct((2 * P, D), jnp.float32),
    mesh=_MESH,
    scratch_types=[
        pltpu.VMEM((GCHUNK,), jnp.int32),
        pltpu.VMEM((GCHUNK, D), jnp.float32),
        pltpu.SemaphoreType.DMA,
    ],
)


def _dinv(dpa, dpb):
    deg = dpa + dpb + 1.0
    return lax.rsqrt(deg)


def _mm_scale_body(dpa_ref, dpb_ref, x_ref, w_ref, o_ref):
    dinv = _dinv(dpa_ref[...], dpb_ref[...])
    h = jnp.dot(x_ref[...], w_ref[...], preferred_element_type=jnp.float32)
    o_ref[...] = h * dinv


def _mid_body(dpa_ref, dpb_ref, pa_ref, pb_ref, hd_ref, b_ref, w_ref, o_ref):
    dinv = _dinv(dpa_ref[...], dpb_ref[...])
    agg = pa_ref[...] + pb_ref[...] + hd_ref[...]
    h1 = jnp.maximum(agg * dinv + b_ref[...], 0.0)
    o_ref[...] = jnp.dot(h1, w_ref[...], preferred_element_type=jnp.float32) * dinv


def _final_body(dpa_ref, dpb_ref, qa_ref, qb_ref, hd_ref, b_ref, o_ref):
    dinv = _dinv(dpa_ref[...], dpb_ref[...])
    agg = qa_ref[...] + qb_ref[...] + hd_ref[...]
    o_ref[...] = agg * dinv + b_ref[...]


BR = 1024


def _full(shape):
    return pl.BlockSpec(shape, lambda i: (0,) * len(shape))


def _row_call(body, n_rows_in, w_shapes, nrows=NPAD):
    in_specs = (
        [pl.BlockSpec((BR, D), lambda i: (i, 0))] * n_rows_in
        + [_full(sh) for sh in w_shapes]
    )
    return pl.pallas_call(
        body,
        grid=(nrows // BR,),
        in_specs=in_specs,
        out_specs=pl.BlockSpec((BR, D), lambda i: (i, 0)),
        out_shape=jax.ShapeDtypeStruct((nrows, D), jnp.float32),
    )


def _rest_body(scg_ref, esm_ref, gpt_ref, wscg_ref, wesm_ref, wgpt_ref, cb1_ref, o_ref):
    z = jnp.dot(scg_ref[...], wscg_ref[...], preferred_element_type=jnp.float32)
    z += jnp.dot(esm_ref[...], wesm_ref[...], preferred_element_type=jnp.float32)
    z += jnp.dot(gpt_ref[...], wgpt_ref[...], preferred_element_type=jnp.float32)
    o_ref[...] = z + cb1_ref[...]


def _mlp_body(ne1_ref, ne2_ref, rest_ref, wg1_ref, wg2_ref,
              w2_ref, cb2_ref, w3_ref, cb3_ref, o_ref):
    z1 = jnp.dot(ne1_ref[...], wg1_ref[...], preferred_element_type=jnp.float32)
    z1 += jnp.dot(ne2_ref[...], wg2_ref[...], preferred_element_type=jnp.float32)
    z1 = jnp.maximum(z1 + rest_ref[...], 0.0)
    z2 = jnp.maximum(
        jnp.dot(z1, w2_ref[...], preferred_element_type=jnp.float32) + cb2_ref[...], 0.0)
    o_ref[...] = jnp.dot(z2, w3_ref[...], preferred_element_type=jnp.float32) + cb3_ref[...]


MBR = 512


def _rest_call(scgw, esmw, gptw):
    in_specs = (
        [pl.BlockSpec((MBR, scgw), lambda i: (i, 0)),
         pl.BlockSpec((MBR, esmw), lambda i: (i, 0)),
         pl.BlockSpec((MBR, gptw), lambda i: (i, 0))]
        + [_full(sh) for sh in [(scgw, D), (esmw, D), (gptw, D), (1, D)]]
    )
    return pl.pallas_call(
        _rest_body,
        grid=(P // MBR,),
        in_specs=in_specs,
        out_specs=pl.BlockSpec((MBR, D), lambda i: (i, 0)),
        out_shape=jax.ShapeDtypeStruct((P, D), jnp.float32),
    )


def _mlp_call():
    in_specs = (
        [pl.BlockSpec((MBR, D), lambda i: (i, 0))] * 3
        + [_full(sh) for sh in [(D, D), (D, D), (D, D), (1, D), (D, D), (1, D)]]
    )
    return pl.pallas_call(
        _mlp_body,
        grid=(P // MBR,),
        in_specs=in_specs,
        out_specs=pl.BlockSpec((MBR, D), lambda i: (i, 0)),
        out_shape=jax.ShapeDtypeStruct((P, D), jnp.float32),
    )


def kernel(x, edge_index, scg_pair, gpt_pair, esm_pair, pair_idx,
           W1, b1, W2, b2, cW1, cb1, cW2, cb2, cW3, cb3):
    f32 = jnp.float32
    x_pad = jnp.pad(x, ((0, NPAD - N), (0, 0)))
    z128 = jnp.zeros((NPAD, D), f32)
    ones128 = jnp.ones((NPAD, D), f32)
    src2 = edge_index[0].reshape(NROW, CH)
    dst2 = edge_index[1].reshape(NROW, CH)

    scgw = scg_pair.shape[1]
    esmw = esm_pair.shape[1]
    gptw = gpt_pair.shape[1]
    wg1 = cW1[:D]
    wg2 = cW1[D:2 * D]
    wscg = cW1[2 * D:2 * D + scgw]
    wesm = cW1[2 * D + scgw:2 * D + scgw + esmw]
    wgpt = cW1[2 * D + scgw + esmw:]
    cW2p = jnp.pad(cW2, ((0, 0), (0, D - cW2.shape[1])))
    cb2p = jnp.pad(cb2, (0, D - cb2.shape[0])).reshape(1, D)
    cW3p = jnp.pad(cW3, ((0, D - cW3.shape[0]), (0, D - cW3.shape[1])))
    cb3p = jnp.pad(cb3, (0, D - cb3.shape[0])).reshape(1, D)

    deg_parts = _agg_call(ones128, dst2, dst2, z128)
    rest = _rest_call(scgw, esmw, gptw)(
        scg_pair, esm_pair, gpt_pair, wscg, wesm, wgpt, cb1.reshape(1, D))

    dpa, dpb = deg_parts[0], deg_parts[1]
    hd1 = _row_call(_mm_scale_body, 3, [(D, D)])(dpa, dpb, x_pad, W1)
    p = _agg_call(hd1, src2, dst2, z128)
    hd2 = _row_call(_mid_body, 5, [(1, D), (D, D)])(
        dpa, dpb, p[0], p[1], hd1, b1.reshape(1, D), W2)
    q = _agg_call(hd2, src2, dst2, z128)
    h2 = _row_call(_final_body, 5, [(1, D)])(
        dpa, dpb, q[0], q[1], hd2, b2.reshape(1, D))

    idx_flat = jnp.transpose(pair_idx).reshape(2 * P)
    pg = _pair_gather_call(h2, idx_flat)
    ne1, ne2 = pg[:P], pg[P:]

    out = _mlp_call()(ne1, ne2, rest, wg1, wg2, cW2p, cb2p, cW3p, cb3p)
    return out[:, :cW3.shape[1]]


# speedup vs baseline: 1.3042x; 1.1040x over previous
import functools

import jax
import jax.numpy as jnp
from jax import lax
from jax.experimental import pallas as pl
from jax.experimental.pallas import tpu as pltpu
from jax.experimental.pallas import tpu_sc as plsc

N = 10000
NPAD = 10240
E = 640000
D = 128
P = 4096

NC, NS = 2, 16
NW = NC * NS
CH = 128
NROW = E // CH
GRP = 8
NGROUP = NROW // GRP
GREM = NGROUP % NW
MAXG = NGROUP // NW + 1
NBUF = 4
RPT = NPAD // NS

_MESH = plsc.VectorSubcoreMesh(
    core_axis_name="c", subcore_axis_name="s", num_cores=NC, num_subcores=NS)


def _worker_groups(wid):
    return jnp.where(wid < GREM, NGROUP // NW + 1, NGROUP // NW)


def _agg_body(hd, src2, dst2, zeros128, out,
              idx_s, idx_d, r0, r1, sem_i, g0, g1, s0, s1, acc):
    cid = lax.axis_index("c")
    sid = lax.axis_index("s")
    wid = sid * NC + cid
    sl = pl.ds(sid * RPT, RPT)
    ng = _worker_groups(wid)

    pltpu.sync_copy(zeros128.at[sl], acc.at[sl])
    gr0 = wid * GRP
    pltpu.sync_copy(src2.at[pl.ds(gr0, GRP)], idx_s.at[pl.ds(0, GRP)])
    pltpu.sync_copy(dst2.at[pl.ds(gr0, GRP)], idx_d.at[pl.ds(0, GRP)])
    plsc.subcore_barrier()
    pltpu.async_copy(hd.at[idx_s.at[0]], r0, g0)

    def group_body(g, c):
        half = (g % 2) * GRP
        nxt_half = ((g + 1) % 2) * GRP
        have_next = g + 1 < ng

        @pl.when(have_next)
        def _prefetch_idx():
            gr = (wid + (g + 1) * NW) * GRP
            pltpu.async_copy(src2.at[pl.ds(gr, GRP)],
                             idx_s.at[pl.ds(nxt_half, GRP)], sem_i)
            pltpu.async_copy(dst2.at[pl.ds(gr, GRP)],
                             idx_d.at[pl.ds(nxt_half, GRP)], sem_i)

        prev_scatter = None
        for j in range(GRP):
            rr, gg = (r0, g0) if j % 2 == 0 else (r1, g1)
            orr, ogg = (r1, g1) if j % 2 == 0 else (r0, g0)
            sc = s0 if j % 2 == 0 else s1
            row = half + j
            if prev_scatter is not None:
                prev_scatter.wait()
            if j < GRP - 1:
                pltpu.async_copy(hd.at[idx_s.at[row + 1]], orr, ogg)
            else:
                @pl.when(have_next)
                def _next_group_gather():
                    pltpu.make_async_copy(
                        src2.at[pl.ds(0, GRP)], idx_s.at[pl.ds(0, GRP)], sem_i).wait()
                    pltpu.make_async_copy(
                        dst2.at[pl.ds(0, GRP)], idx_d.at[pl.ds(0, GRP)], sem_i).wait()
                    pltpu.async_copy(hd.at[idx_s.at[nxt_half]], orr, ogg)
            pltpu.make_async_copy(zeros128.at[pl.ds(0, CH)], rr, gg).wait()
            if j < GRP - 1:
                prev_scatter = pltpu.async_copy(
                    rr, acc.at[idx_d.at[row]], sc, add=True)
            else:
                pltpu.sync_copy(rr, acc.at[idx_d.at[row]], add=True)
        return c

    lax.fori_loop(0, ng, group_body, 0)
    plsc.subcore_barrier()
    pltpu.sync_copy(acc.at[sl], out.at[cid, sl])


_agg_call = pl.kernel(
    _agg_body,
    out_type=jax.ShapeDtypeStruct((NC, NPAD, D), jnp.float32),
    mesh=_MESH,
    scratch_types=[
        pltpu.VMEM((2 * GRP, CH), jnp.int32),
        pltpu.VMEM((2 * GRP, CH), jnp.int32),
        pltpu.VMEM((CH, D), jnp.float32),
        pltpu.VMEM((CH, D), jnp.float32),
        pltpu.SemaphoreType.DMA,
        pltpu.SemaphoreType.DMA,
        pltpu.SemaphoreType.DMA,
        pltpu.SemaphoreType.DMA,
        pltpu.SemaphoreType.DMA,
        pltpu.VMEM_SHARED((NPAD, D), jnp.float32),
    ],
)

def _deg_body(ones128, dst2, zeros128, out, idx_d, r0, sem_i, s0, s1, acc):
    cid = lax.axis_index("c")
    sid = lax.axis_index("s")
    wid = sid * NC + cid
    sl = pl.ds(sid * RPT, RPT)
    ng = _worker_groups(wid)

    pltpu.sync_copy(zeros128.at[sl], acc.at[sl])
    gr0 = wid * GRP
    pltpu.sync_copy(dst2.at[pl.ds(gr0, GRP)], idx_d.at[pl.ds(0, GRP)])
    pltpu.sync_copy(ones128.at[pl.ds(0, CH)], r0)
    plsc.subcore_barrier()

    def group_body(g, c):
        half = (g % 2) * GRP
        nxt_half = ((g + 1) % 2) * GRP
        have_next = g + 1 < ng

        @pl.when(have_next)
        def _prefetch_idx():
            gr = (wid + (g + 1) * NW) * GRP
            pltpu.async_copy(dst2.at[pl.ds(gr, GRP)],
                             idx_d.at[pl.ds(nxt_half, GRP)], sem_i)

        prev_scatter = None
        for j in range(GRP):
            sc = s0 if j % 2 == 0 else s1
            row = half + j
            if prev_scatter is not None:
                prev_scatter.wait()
            if j < GRP - 1:
                prev_scatter = pltpu.async_copy(
                    r0, acc.at[idx_d.at[row]], sc, add=True)
            else:
                @pl.when(have_next)
                def _wait_idx():
                    pltpu.make_async_copy(
                        dst2.at[pl.ds(0, GRP)], idx_d.at[pl.ds(0, GRP)], sem_i).wait()
                pltpu.sync_copy(r0, acc.at[idx_d.at[row]], add=True)
        return c

    lax.fori_loop(0, ng, group_body, 0)
    plsc.subcore_barrier()
    pltpu.sync_copy(acc.at[sl], out.at[cid, sl])


_deg_call = pl.kernel(
    _deg_body,
    out_type=jax.ShapeDtypeStruct((NC, NPAD, D), jnp.float32),
    mesh=_MESH,
    scratch_types=[
        pltpu.VMEM((2 * GRP, CH), jnp.int32),
        pltpu.VMEM((CH, D), jnp.float32),
        pltpu.SemaphoreType.DMA,
        pltpu.SemaphoreType.DMA,
        pltpu.SemaphoreType.DMA,
        pltpu.VMEM_SHARED((NPAD, D), jnp.float32),
    ],
)


GCHUNK = 128
GPW = 2 * P // NW


def _pair_gather_body(h2, idxflat, out, gidx, grows, sem):
    cid = lax.axis_index("c")
    sid = lax.axis_index("s")
    wid = sid * NC + cid
    base0 = wid * GPW

    def body(i, carry):
        base = base0 + i * GCHUNK
        pltpu.sync_copy(idxflat.at[pl.ds(base, GCHUNK)], gidx)
        pltpu.async_copy(h2.at[gidx], grows, sem).wait()
        pltpu.sync_copy(grows, out.at[pl.ds(base, GCHUNK)])
        return carry

    lax.fori_loop(0, GPW // GCHUNK, body, 0)


_pair_gather_call = pl.kernel(
    _pair_gather_body,
    out_type=jax.ShapeDtypeStruct((2 * P, D), jnp.float32),
    mesh=_MESH,
    scratch_types=[
        pltpu.VMEM((GCHUNK,), jnp.int32),
        pltpu.VMEM((GCHUNK, D), jnp.float32),
        pltpu.SemaphoreType.DMA,
    ],
)


def _dinv(dpa, dpb):
    deg = dpa + dpb + 1.0
    return lax.rsqrt(deg)


def _mm_scale_body(dpa_ref, dpb_ref, x_ref, w_ref, o_ref):
    dinv = _dinv(dpa_ref[...], dpb_ref[...])
    h = jnp.dot(x_ref[...], w_ref[...], preferred_element_type=jnp.float32)
    o_ref[...] = h * dinv


def _mid_body(dpa_ref, dpb_ref, pa_ref, pb_ref, hd_ref, b_ref, w_ref, o_ref):
    dinv = _dinv(dpa_ref[...], dpb_ref[...])
    agg = pa_ref[...] + pb_ref[...] + hd_ref[...]
    h1 = jnp.maximum(agg * dinv + b_ref[...], 0.0)
    o_ref[...] = jnp.dot(h1, w_ref[...], preferred_element_type=jnp.float32) * dinv


def _final_body(dpa_ref, dpb_ref, qa_ref, qb_ref, hd_ref, b_ref, o_ref):
    dinv = _dinv(dpa_ref[...], dpb_ref[...])
    agg = qa_ref[...] + qb_ref[...] + hd_ref[...]
    o_ref[...] = agg * dinv + b_ref[...]


BR = 1024


def _full(shape):
    return pl.BlockSpec(shape, lambda i: (0,) * len(shape))


def _row_call(body, n_rows_in, w_shapes, nrows=NPAD):
    in_specs = (
        [pl.BlockSpec((BR, D), lambda i: (i, 0))] * n_rows_in
        + [_full(sh) for sh in w_shapes]
    )
    return pl.pallas_call(
        body,
        grid=(nrows // BR,),
        in_specs=in_specs,
        out_specs=pl.BlockSpec((BR, D), lambda i: (i, 0)),
        out_shape=jax.ShapeDtypeStruct((nrows, D), jnp.float32),
    )


def _rest_body(scg_ref, esm_ref, gpt_ref, wscg_ref, wesm_ref, wgpt_ref, cb1_ref, o_ref):
    z = jnp.dot(scg_ref[...], wscg_ref[...], preferred_element_type=jnp.float32)
    z += jnp.dot(esm_ref[...], wesm_ref[...], preferred_element_type=jnp.float32)
    z += jnp.dot(gpt_ref[...], wgpt_ref[...], preferred_element_type=jnp.float32)
    o_ref[...] = z + cb1_ref[...]


def _mlp_body(ne1_ref, ne2_ref, rest_ref, wg1_ref, wg2_ref,
              w2_ref, cb2_ref, w3_ref, cb3_ref, o_ref):
    z1 = jnp.dot(ne1_ref[...], wg1_ref[...], preferred_element_type=jnp.float32)
    z1 += jnp.dot(ne2_ref[...], wg2_ref[...], preferred_element_type=jnp.float32)
    z1 = jnp.maximum(z1 + rest_ref[...], 0.0)
    z2 = jnp.maximum(
        jnp.dot(z1, w2_ref[...], preferred_element_type=jnp.float32) + cb2_ref[...], 0.0)
    o_ref[...] = jnp.dot(z2, w3_ref[...], preferred_element_type=jnp.float32) + cb3_ref[...]


MBR = 512


def _rest_call(scgw, esmw, gptw):
    in_specs = (
        [pl.BlockSpec((MBR, scgw), lambda i: (i, 0)),
         pl.BlockSpec((MBR, esmw), lambda i: (i, 0)),
         pl.BlockSpec((MBR, gptw), lambda i: (i, 0))]
        + [_full(sh) for sh in [(scgw, D), (esmw, D), (gptw, D), (1, D)]]
    )
    return pl.pallas_call(
        _rest_body,
        grid=(P // MBR,),
        in_specs=in_specs,
        out_specs=pl.BlockSpec((MBR, D), lambda i: (i, 0)),
        out_shape=jax.ShapeDtypeStruct((P, D), jnp.float32),
    )


def _mlp_call():
    in_specs = (
        [pl.BlockSpec((MBR, D), lambda i: (i, 0))] * 3
        + [_full(sh) for sh in [(D, D), (D, D), (D, D), (1, D), (D, D), (1, D)]]
    )
    return pl.pallas_call(
        _mlp_body,
        grid=(P // MBR,),
        in_specs=in_specs,
        out_specs=pl.BlockSpec((MBR, D), lambda i: (i, 0)),
        out_shape=jax.ShapeDtypeStruct((P, D), jnp.float32),
    )


def kernel(x, edge_index, scg_pair, gpt_pair, esm_pair, pair_idx,
           W1, b1, W2, b2, cW1, cb1, cW2, cb2, cW3, cb3):
    f32 = jnp.float32
    x_pad = jnp.pad(x, ((0, NPAD - N), (0, 0)))
    z128 = jnp.zeros((NPAD, D), f32)
    ones128 = jnp.ones((NPAD, D), f32)
    src2 = edge_index[0].reshape(NROW, CH)
    dst2 = edge_index[1].reshape(NROW, CH)

    scgw = scg_pair.shape[1]
    esmw = esm_pair.shape[1]
    gptw = gpt_pair.shape[1]
    wg1 = cW1[:D]
    wg2 = cW1[D:2 * D]
    wscg = cW1[2 * D:2 * D + scgw]
    wesm = cW1[2 * D + scgw:2 * D + scgw + esmw]
    wgpt = cW1[2 * D + scgw + esmw:]
    cW2p = jnp.pad(cW2, ((0, 0), (0, D - cW2.shape[1])))
    cb2p = jnp.pad(cb2, (0, D - cb2.shape[0])).reshape(1, D)
    cW3p = jnp.pad(cW3, ((0, D - cW3.shape[0]), (0, D - cW3.shape[1])))
    cb3p = jnp.pad(cb3, (0, D - cb3.shape[0])).reshape(1, D)

    deg_parts = _deg_call(ones128, dst2, z128)
    rest = _rest_call(scgw, esmw, gptw)(
        scg_pair, esm_pair, gpt_pair, wscg, wesm, wgpt, cb1.reshape(1, D))

    dpa, dpb = deg_parts[0], deg_parts[1]
    hd1 = _row_call(_mm_scale_body, 3, [(D, D)])(dpa, dpb, x_pad, W1)
    p = _agg_call(hd1, src2, dst2, z128)
    hd2 = _row_call(_mid_body, 5, [(1, D), (D, D)])(
        dpa, dpb, p[0], p[1], hd1, b1.reshape(1, D), W2)
    q = _agg_call(hd2, src2, dst2, z128)
    h2 = _row_call(_final_body, 5, [(1, D)])(
        dpa, dpb, q[0], q[1], hd2, b2.reshape(1, D))

    idx_flat = jnp.transpose(pair_idx).reshape(2 * P)
    pg = _pair_gather_call(h2, idx_flat)
    ne1, ne2 = pg[:P], pg[P:]

    out = _mlp_call()(ne1, ne2, rest, wg1, wg2, cW2p, cb2p, cW3p, cb3p)
    return out[:, :cW3.shape[1]]


# speedup vs baseline: 1.3043x; 1.0001x over previous
import jax
import jax.numpy as jnp
from jax import lax
from jax.experimental import pallas as pl
from jax.experimental.pallas import tpu as pltpu
from jax.experimental.pallas import tpu_sc as plsc

N = 10000
NPAD = 10240
E = 640000
D = 128
P = 4096

NC, NS = 2, 16
NW = NC * NS
CH = 128
NROW = E // CH
GRP = 8
NGROUP = NROW // GRP
GREM = NGROUP % NW
MAXG = NGROUP // NW + 1
RPT = NPAD // NS

_MESH = plsc.VectorSubcoreMesh(
    core_axis_name="c", subcore_axis_name="s", num_cores=NC, num_subcores=NS)


def _worker_groups(wid):
    return jnp.where(wid < GREM, NGROUP // NW + 1, NGROUP // NW)


def _agg_body(hd, src2, dst2, zeros128, out,
              idx_s, idx_d, r0, r1, sem_i, g0, g1, s0, s1, acc):
    cid = lax.axis_index("c")
    sid = lax.axis_index("s")
    wid = sid * NC + cid
    sl = pl.ds(sid * RPT, RPT)
    ng = _worker_groups(wid)

    pltpu.sync_copy(zeros128.at[sl], acc.at[sl])
    gr0 = wid * GRP
    pltpu.sync_copy(src2.at[pl.ds(gr0, GRP)], idx_s.at[pl.ds(0, GRP)])
    pltpu.sync_copy(dst2.at[pl.ds(gr0, GRP)], idx_d.at[pl.ds(0, GRP)])
    plsc.subcore_barrier()
    pltpu.async_copy(hd.at[idx_s.at[0]], r0, g0)

    def group_body(g, c):
        half = (g % 2) * GRP
        nxt_half = ((g + 1) % 2) * GRP
        have_next = g + 1 < ng

        @pl.when(have_next)
        def _prefetch_idx():
            gr = (wid + (g + 1) * NW) * GRP
            pltpu.async_copy(src2.at[pl.ds(gr, GRP)],
                             idx_s.at[pl.ds(nxt_half, GRP)], sem_i)
            pltpu.async_copy(dst2.at[pl.ds(gr, GRP)],
                             idx_d.at[pl.ds(nxt_half, GRP)], sem_i)

        prev_scatter = None
        for j in range(GRP):
            rr, gg = (r0, g0) if j % 2 == 0 else (r1, g1)
            orr, ogg = (r1, g1) if j % 2 == 0 else (r0, g0)
            sc = s0 if j % 2 == 0 else s1
            row = half + j
            if prev_scatter is not None:
                prev_scatter.wait()
            if j < GRP - 1:
                pltpu.async_copy(hd.at[idx_s.at[row + 1]], orr, ogg)
            else:
                @pl.when(have_next)
                def _next_group_gather():
                    pltpu.make_async_copy(
                        src2.at[pl.ds(0, GRP)], idx_s.at[pl.ds(0, GRP)], sem_i).wait()
                    pltpu.make_async_copy(
                        dst2.at[pl.ds(0, GRP)], idx_d.at[pl.ds(0, GRP)], sem_i).wait()
                    pltpu.async_copy(hd.at[idx_s.at[nxt_half]], orr, ogg)
            pltpu.make_async_copy(zeros128.at[pl.ds(0, CH)], rr, gg).wait()
            if j < GRP - 1:
                prev_scatter = pltpu.async_copy(
                    rr, acc.at[idx_d.at[row]], sc, add=True)
            else:
                pltpu.sync_copy(rr, acc.at[idx_d.at[row]], add=True)
        return c

    lax.fori_loop(0, ng, group_body, 0)
    plsc.subcore_barrier()
    pltpu.sync_copy(acc.at[sl], out.at[cid, sl])


_agg_call = pl.kernel(
    _agg_body,
    out_type=jax.ShapeDtypeStruct((NC, NPAD, D), jnp.float32),
    mesh=_MESH,
    scratch_types=[
        pltpu.VMEM((2 * GRP, CH), jnp.int32),
        pltpu.VMEM((2 * GRP, CH), jnp.int32),
        pltpu.VMEM((CH, D), jnp.float32),
        pltpu.VMEM((CH, D), jnp.float32),
        pltpu.SemaphoreType.DMA,
        pltpu.SemaphoreType.DMA,
        pltpu.SemaphoreType.DMA,
        pltpu.SemaphoreType.DMA,
        pltpu.SemaphoreType.DMA,
        pltpu.VMEM_SHARED((NPAD, D), jnp.float32),
    ],
)

def _deg_body(ones128, dst2, zeros128, out, idx_d, r0, sem_i, s0, s1, acc):
    cid = lax.axis_index("c")
    sid = lax.axis_index("s")
    wid = sid * NC + cid
    sl = pl.ds(sid * RPT, RPT)
    ng = _worker_groups(wid)

    pltpu.sync_copy(zeros128.at[sl], acc.at[sl])
    gr0 = wid * GRP
    pltpu.sync_copy(dst2.at[pl.ds(gr0, GRP)], idx_d.at[pl.ds(0, GRP)])
    pltpu.sync_copy(ones128.at[pl.ds(0, CH)], r0)
    plsc.subcore_barrier()

    def group_body(g, c):
        half = (g % 2) * GRP
        nxt_half = ((g + 1) % 2) * GRP
        have_next = g + 1 < ng

        @pl.when(have_next)
        def _prefetch_idx():
            gr = (wid + (g + 1) * NW) * GRP
            pltpu.async_copy(dst2.at[pl.ds(gr, GRP)],
                             idx_d.at[pl.ds(nxt_half, GRP)], sem_i)

        prev_scatter = None
        for j in range(GRP):
            sc = s0 if j % 2 == 0 else s1
            row = half + j
            if prev_scatter is not None:
                prev_scatter.wait()
            if j < GRP - 1:
                prev_scatter = pltpu.async_copy(
                    r0, acc.at[idx_d.at[row]], sc, add=True)
            else:
                @pl.when(have_next)
                def _wait_idx():
                    pltpu.make_async_copy(
                        dst2.at[pl.ds(0, GRP)], idx_d.at[pl.ds(0, GRP)], sem_i).wait()
                pltpu.sync_copy(r0, acc.at[idx_d.at[row]], add=True)
        return c

    lax.fori_loop(0, ng, group_body, 0)
    plsc.subcore_barrier()
    pltpu.sync_copy(acc.at[sl], out.at[cid, sl])


_deg_call = pl.kernel(
    _deg_body,
    out_type=jax.ShapeDtypeStruct((NC, NPAD, D), jnp.float32),
    mesh=_MESH,
    scratch_types=[
        pltpu.VMEM((2 * GRP, CH), jnp.int32),
        pltpu.VMEM((CH, D), jnp.float32),
        pltpu.SemaphoreType.DMA,
        pltpu.SemaphoreType.DMA,
        pltpu.SemaphoreType.DMA,
        pltpu.VMEM_SHARED((NPAD, D), jnp.float32),
    ],
)


GCHUNK = 128
GPW = 2 * P // NW


def _pair_gather_body(h2, idxflat, out, gidx, grows, sem):
    cid = lax.axis_index("c")
    sid = lax.axis_index("s")
    wid = sid * NC + cid
    base0 = wid * GPW

    def body(i, carry):
        base = base0 + i * GCHUNK
        pltpu.sync_copy(idxflat.at[pl.ds(base, GCHUNK)], gidx)
        pltpu.async_copy(h2.at[gidx], grows, sem).wait()
        pltpu.sync_copy(grows, out.at[pl.ds(base, GCHUNK)])
        return carry

    lax.fori_loop(0, GPW // GCHUNK, body, 0)


_pair_gather_call = pl.kernel(
    _pair_gather_body,
    out_type=jax.ShapeDtypeStruct((2 * P, D), jnp.float32),
    mesh=_MESH,
    scratch_types=[
        pltpu.VMEM((GCHUNK,), jnp.int32),
        pltpu.VMEM((GCHUNK, D), jnp.float32),
        pltpu.SemaphoreType.DMA,
    ],
)


def _dinv(dpa, dpb):
    deg = dpa + dpb + 1.0
    return lax.rsqrt(deg)


def _mm_scale_body(dpa_ref, dpb_ref, x_ref, w_ref, o_ref):
    dinv = _dinv(dpa_ref[...], dpb_ref[...])
    h = jnp.dot(x_ref[...], w_ref[...], preferred_element_type=jnp.float32)
    o_ref[...] = h * dinv


def _mid_body(dpa_ref, dpb_ref, pa_ref, pb_ref, hd_ref, b_ref, w_ref, o_ref):
    dinv = _dinv(dpa_ref[...], dpb_ref[...])
    agg = pa_ref[...] + pb_ref[...] + hd_ref[...]
    h1 = jnp.maximum(agg * dinv + b_ref[...], 0.0)
    o_ref[...] = jnp.dot(h1, w_ref[...], preferred_element_type=jnp.float32) * dinv


def _final_body(dpa_ref, dpb_ref, qa_ref, qb_ref, hd_ref, b_ref, o_ref):
    dinv = _dinv(dpa_ref[...], dpb_ref[...])
    agg = qa_ref[...] + qb_ref[...] + hd_ref[...]
    o_ref[...] = agg * dinv + b_ref[...]


BR = 1024


def _full(shape):
    return pl.BlockSpec(shape, lambda i: (0,) * len(shape))


def _row_call(body, n_rows_in, w_shapes, nrows=NPAD):
    in_specs = (
        [pl.BlockSpec((BR, D), lambda i: (i, 0))] * n_rows_in
        + [_full(sh) for sh in w_shapes]
    )
    return pl.pallas_call(
        body,
        grid=(nrows // BR,),
        in_specs=in_specs,
        out_specs=pl.BlockSpec((BR, D), lambda i: (i, 0)),
        out_shape=jax.ShapeDtypeStruct((nrows, D), jnp.float32),
    )


def _rest_body(scg_ref, esm_ref, gpt_ref, wscg_ref, wesm_ref, wgpt_ref, cb1_ref, o_ref):
    z = jnp.dot(scg_ref[...], wscg_ref[...], preferred_element_type=jnp.float32)
    z += jnp.dot(esm_ref[...], wesm_ref[...], preferred_element_type=jnp.float32)
    z += jnp.dot(gpt_ref[...], wgpt_ref[...], preferred_element_type=jnp.float32)
    o_ref[...] = z + cb1_ref[...]


def _mlp_body(ne1_ref, ne2_ref, rest_ref, wg1_ref, wg2_ref,
              w2_ref, cb2_ref, w3_ref, cb3_ref, o_ref):
    z1 = jnp.dot(ne1_ref[...], wg1_ref[...], preferred_element_type=jnp.float32)
    z1 += jnp.dot(ne2_ref[...], wg2_ref[...], preferred_element_type=jnp.float32)
    z1 = jnp.maximum(z1 + rest_ref[...], 0.0)
    z2 = jnp.maximum(
        jnp.dot(z1, w2_ref[...], preferred_element_type=jnp.float32) + cb2_ref[...], 0.0)
    o_ref[...] = jnp.dot(z2, w3_ref[...], preferred_element_type=jnp.float32) + cb3_ref[...]


MBR = 512


def _rest_call(scgw, esmw, gptw):
    in_specs = (
        [pl.BlockSpec((MBR, scgw), lambda i: (i, 0)),
         pl.BlockSpec((MBR, esmw), lambda i: (i, 0)),
         pl.BlockSpec((MBR, gptw), lambda i: (i, 0))]
        + [_full(sh) for sh in [(scgw, D), (esmw, D), (gptw, D), (1, D)]]
    )
    return pl.pallas_call(
        _rest_body,
        grid=(P // MBR,),
        in_specs=in_specs,
        out_specs=pl.BlockSpec((MBR, D), lambda i: (i, 0)),
        out_shape=jax.ShapeDtypeStruct((P, D), jnp.float32),
    )


def _mlp_call():
    in_specs = (
        [pl.BlockSpec((MBR, D), lambda i: (i, 0))] * 3
        + [_full(sh) for sh in [(D, D), (D, D), (D, D), (1, D), (D, D), (1, D)]]
    )
    return pl.pallas_call(
        _mlp_body,
        grid=(P // MBR,),
        in_specs=in_specs,
        out_specs=pl.BlockSpec((MBR, D), lambda i: (i, 0)),
        out_shape=jax.ShapeDtypeStruct((P, D), jnp.float32),
    )


def kernel(x, edge_index, scg_pair, gpt_pair, esm_pair, pair_idx,
           W1, b1, W2, b2, cW1, cb1, cW2, cb2, cW3, cb3):
    f32 = jnp.float32
    x_pad = jnp.pad(x, ((0, NPAD - N), (0, 0)))
    z128 = jnp.zeros((NPAD, D), f32)
    ones128 = jnp.ones((NPAD, D), f32)
    src2 = edge_index[0].reshape(NROW, CH)
    dst2 = edge_index[1].reshape(NROW, CH)

    scgw = scg_pair.shape[1]
    esmw = esm_pair.shape[1]
    gptw = gpt_pair.shape[1]
    wg1 = cW1[:D]
    wg2 = cW1[D:2 * D]
    wscg = cW1[2 * D:2 * D + scgw]
    wesm = cW1[2 * D + scgw:2 * D + scgw + esmw]
    wgpt = cW1[2 * D + scgw + esmw:]
    cW2p = jnp.pad(cW2, ((0, 0), (0, D - cW2.shape[1])))
    cb2p = jnp.pad(cb2, (0, D - cb2.shape[0])).reshape(1, D)
    cW3p = jnp.pad(cW3, ((0, D - cW3.shape[0]), (0, D - cW3.shape[1])))
    cb3p = jnp.pad(cb3, (0, D - cb3.shape[0])).reshape(1, D)

    deg_parts = _deg_call(ones128, dst2, z128)
    rest = _rest_call(scgw, esmw, gptw)(
        scg_pair, esm_pair, gpt_pair, wscg, wesm, wgpt, cb1.reshape(1, D))

    dpa, dpb = deg_parts[0], deg_parts[1]
    hd1 = _row_call(_mm_scale_body, 3, [(D, D)])(dpa, dpb, x_pad, W1)
    p = _agg_call(hd1, src2, dst2, z128)
    hd2 = _row_call(_mid_body, 5, [(1, D), (D, D)])(
        dpa, dpb, p[0], p[1], hd1, b1.reshape(1, D), W2)
    q = _agg_call(hd2, src2, dst2, z128)
    h2 = _row_call(_final_body, 5, [(1, D)])(
        dpa, dpb, q[0], q[1], hd2, b2.reshape(1, D))

    idx_flat = jnp.transpose(pair_idx).reshape(2 * P)
    pg = _pair_gather_call(h2, idx_flat)
    ne1, ne2 = pg[:P], pg[P:]

    out = _mlp_call()(ne1, ne2, rest, wg1, wg2, cW2p, cb2p, cW3p, cb3p)
    return out[:, :cW3.shape[1]]
